# Initial kernel scaffold; baseline (speedup 1.0000x reference)
import jax
import jax.numpy as jnp
from jax.experimental import pallas as pl


def kernel(x, edge_index, W_root, W_neigh, b_conv, W1, b1, W2, b2, W3, b3):
    raise NotImplementedError("write your pallas kernel here")


# speedup vs baseline: 3.4217x; 3.4217x over previous
import functools

import jax
import jax.numpy as jnp
from jax import lax
from jax.experimental import pallas as pl
from jax.experimental.pallas import tpu as pltpu
from jax.experimental.pallas import tpu_sc as plsc

_N = 10000
_E = 160000
_D = 256
_H = 512

_NSUB = 16
_CHUNK = 128
_NCH = 79
_EPAD = _NSUB * _NCH * _CHUNK
_TW = 144
_NT = 10016
_RPS = _N // _NSUB


def _sc_aggregate(table, srcp, dstp):
    mesh = plsc.VectorSubcoreMesh(core_axis_name="c", subcore_axis_name="s")

    @functools.partial(
        pl.kernel,
        out_type=jax.ShapeDtypeStruct((2, _N, _TW), jnp.float32),
        mesh=mesh,
        scratch_types=[
            pltpu.VMEM((_CHUNK,), jnp.int32),
            pltpu.VMEM((_CHUNK,), jnp.int32),
            pltpu.VMEM((_CHUNK, _TW), jnp.float32),
            pltpu.VMEM_SHARED((_N, _TW), jnp.float32),
            pltpu.SemaphoreType.DMA,
        ],
        compiler_params=pltpu.CompilerParams(use_tc_tiling_on_sc=False),
    )
    def k(table_hbm, src_hbm, dst_hbm, out_hbm, src_v, dst_v, rows_v, acc, sem):
        c = lax.axis_index("c")
        s = lax.axis_index("s")

        zv = jnp.zeros((1, 16), jnp.float32)

        @pl.loop(0, _CHUNK)
        def _(r):
            @pl.loop(0, _TW, step=16)
            def _(cc):
                rows_v[pl.ds(r, 1), pl.ds(cc, 16)] = zv

        @pl.loop(0, 5)
        def _(kk):
            pltpu.sync_copy(rows_v.at[pl.ds(0, 125)],
                            acc.at[pl.ds(s * _RPS + kk * 125, 125)])

        plsc.subcore_barrier()

        base0 = s * (_NCH * _CHUNK)

        @pl.loop(0, _NCH)
        def _(g):
            b = base0 + g * _CHUNK
            pltpu.sync_copy(src_hbm.at[pl.ds(b, _CHUNK)], src_v)
            pltpu.sync_copy(dst_hbm.at[pl.ds(b, _CHUNK)], dst_v)
            pltpu.async_copy(table_hbm.at[c].at[src_v], rows_v, sem).wait()
            pltpu.sync_copy(rows_v, acc.at[dst_v], add=True)

        plsc.subcore_barrier()

        @pl.loop(0, 5)
        def _(kk):
            r0 = s * _RPS + kk * 125
            pltpu.sync_copy(acc.at[pl.ds(r0, 125)],
                            out_hbm.at[c].at[pl.ds(r0, 125)])

    return k(table, srcp, dstp)


def _mlp_body(x_ref, agg_ref, wr_ref, wn_ref, bc_ref, w1_ref, b1_ref,
              w2_ref, b2_ref, w3_ref, b3_ref, o_ref):
    a = agg_ref[0]
    b = agg_ref[1]
    inv = 1.0 / jnp.maximum(a[:, 128:129], 1.0)
    na = a[:, :128] * inv
    nb = b[:, :128] * inv
    f32 = jnp.float32
    h = jnp.dot(x_ref[...], wr_ref[...], preferred_element_type=f32)
    h = h + jnp.dot(na, wn_ref[:128], preferred_element_type=f32)
    h = h + jnp.dot(nb, wn_ref[128:], preferred_element_type=f32)
    h = jnp.maximum(h + bc_ref[...], 0.0)
    z = jnp.maximum(jnp.dot(h, w1_ref[...], preferred_element_type=f32)
                    + b1_ref[...], 0.0)
    z = jnp.maximum(jnp.dot(z, w2_ref[...], preferred_element_type=f32)
                    + b2_ref[...], 0.0)
    l = jnp.dot(z, w3_ref[...], preferred_element_type=f32) + b3_ref[...]
    m = jnp.max(l, axis=-1, keepdims=True)
    e = jnp.exp(l - m)
    o_ref[...] = e / jnp.sum(e, axis=-1, keepdims=True)


def _mlp(x, agg, W_root, W_neigh, b_conv, W1, b1, W2, b2, W3, b3):
    B = 2000
    grid = (_N // B,)
    full = lambda shape: pl.BlockSpec(shape, lambda i: tuple(0 for _ in shape))
    return pl.pallas_call(
        _mlp_body,
        grid=grid,
        in_specs=[
            pl.BlockSpec((B, _D), lambda i: (i, 0)),
            pl.BlockSpec((2, B, _TW), lambda i: (0, i, 0)),
            full((_D, _H)),
            full((_D, _H)),
            full((1, _H)),
            full((_H, 400)),
            full((1, 400)),
            full((400, 400)),
            full((1, 400)),
            full((400, 2)),
            full((1, 2)),
        ],
        out_specs=pl.BlockSpec((B, 2), lambda i: (i, 0)),
        out_shape=jax.ShapeDtypeStruct((_N, 2), jnp.float32),
    )(x, agg, W_root, W_neigh, b_conv.reshape(1, _H), W1, b1.reshape(1, 400),
      W2, b2.reshape(1, 400), W3, b3.reshape(1, 2))


@jax.jit
def kernel(x, edge_index, W_root, W_neigh, b_conv, W1, b1, W2, b2, W3, b3):
    ones = jnp.ones((_N, 16), jnp.float32)
    ta = jnp.concatenate([x[:, :128], ones], axis=1)
    tb = jnp.concatenate([x[:, 128:], ones], axis=1)
    table = jnp.zeros((2, _NT, _TW), jnp.float32).at[:, :_N].set(
        jnp.stack([ta, tb]))
    pad = _EPAD - _E
    srcp = jnp.concatenate([edge_index[0], jnp.full((pad,), _N, jnp.int32)])
    dstp = jnp.concatenate([edge_index[1], jnp.zeros((pad,), jnp.int32)])
    agg = _sc_aggregate(table, srcp, dstp)
    return _mlp(x, agg, W_root, W_neigh, b_conv, W1, b1, W2, b2, W3, b3)


# speedup vs baseline: 4.0438x; 1.1818x over previous
import functools

import jax
import jax.numpy as jnp
from jax import lax
from jax.experimental import pallas as pl
from jax.experimental.pallas import tpu as pltpu
from jax.experimental.pallas import tpu_sc as plsc

_N = 10000
_E = 160000
_D = 256
_H = 512

_NSUB = 16
_CHUNK = 128
_NCH = 80
_EPS = _NCH * _CHUNK
_EPAD = _NSUB * _EPS
_NA = _N + _NSUB
_TW = 144


def _sc_aggregate(table, srcp, dstp):
    mesh = plsc.VectorSubcoreMesh(core_axis_name="c", subcore_axis_name="s")

    @functools.partial(
        pl.kernel,
        out_type=jax.ShapeDtypeStruct((2, _N, _TW), jnp.float32),
        mesh=mesh,
        scratch_types=[
            pltpu.VMEM((2, _CHUNK), jnp.int32),
            pltpu.VMEM((2, _CHUNK), jnp.int32),
            pltpu.VMEM((_CHUNK, _TW), jnp.float32),
            pltpu.VMEM((_CHUNK, _TW), jnp.float32),
            pltpu.VMEM_SHARED((_NA, _TW), jnp.float32),
            pltpu.SemaphoreType.DMA,
            pltpu.SemaphoreType.DMA,
        ],
        compiler_params=pltpu.CompilerParams(use_tc_tiling_on_sc=False),
    )
    def k(table_hbm, idx_hbm, agg_hbm, idx_a, idx_b, buf_a, buf_b,
          acc, sem_g, sem_i):
        c = lax.axis_index("c")
        s = lax.axis_index("s")
        idx = idx_hbm.at[s]

        zv = jnp.zeros((1, 16), jnp.float32)

        @pl.loop(0, _CHUNK)
        def _(r):
            @pl.loop(0, _TW, step=16)
            def _(cc):
                buf_a[pl.ds(r, 1), pl.ds(cc, 16)] = zv

        @pl.loop(0, 5)
        def _(kk):
            ch = s + _NSUB * kk

            @pl.when(ch < _NA // _CHUNK)
            def _():
                pltpu.sync_copy(buf_a, acc.at[pl.ds(ch * _CHUNK, _CHUNK)])

            @pl.when(ch == _NA // _CHUNK)
            def _():
                r0 = ch * _CHUNK
                nr = _NA - r0
                pltpu.sync_copy(buf_a.at[pl.ds(0, nr)], acc.at[pl.ds(r0, nr)])

        plsc.subcore_barrier()

        tab = table_hbm.at[c]
        pltpu.sync_copy(idx.at[0], idx_a)
        pltpu.async_copy(tab.at[idx_a.at[0]], buf_a, sem_g)
        pltpu.async_copy(idx.at[1], idx_b, sem_i)

        @pl.loop(0, _NCH // 2)
        def _(g2):
            g = 2 * g2
            pltpu.make_async_copy(idx.at[g + 1], idx_b, sem_i).wait()
            pltpu.make_async_copy(tab.at[idx_a.at[0]], buf_a, sem_g).wait()
            pltpu.async_copy(tab.at[idx_b.at[0]], buf_b, sem_g)
            pltpu.sync_copy(buf_a, acc.at[idx_a.at[1]], add=True)

            @pl.when(g2 < _NCH // 2 - 1)
            def _():
                pltpu.async_copy(idx.at[g + 2], idx_a, sem_i)
                pltpu.make_async_copy(tab.at[idx_b.at[0]], buf_b, sem_g).wait()
                pltpu.make_async_copy(idx.at[g + 2], idx_a, sem_i).wait()
                pltpu.async_copy(tab.at[idx_a.at[0]], buf_a, sem_g)
                pltpu.sync_copy(buf_b, acc.at[idx_b.at[1]], add=True)
                pltpu.async_copy(idx.at[g + 3], idx_b, sem_i)

            @pl.when(g2 == _NCH // 2 - 1)
            def _():
                pltpu.make_async_copy(tab.at[idx_b.at[0]], buf_b, sem_g).wait()
                pltpu.sync_copy(buf_b, acc.at[idx_b.at[1]], add=True)

        plsc.subcore_barrier()

        @pl.loop(0, 8)
        def _(kk):
            ch = s + _NSUB * kk

            @pl.when(ch < _N // 80)
            def _():
                r0 = ch * 80
                pltpu.sync_copy(acc.at[pl.ds(r0, 80)],
                                agg_hbm.at[c].at[pl.ds(r0, 80)])

    return k(table, jnp.stack([srcp, dstp], axis=2))


def _mlp_body(x_ref, agg_ref, wr_ref, wn_ref, bc_ref, w1_ref, b1_ref,
              w2_ref, b2_ref, w3_ref, b3_ref, o_ref):
    a = agg_ref[0]
    inv = 1.0 / jnp.maximum(a[:, 128:129], 1.0)
    na = a[:, :128] * inv
    nb = agg_ref[1][:, :128] * inv
    f32 = jnp.float32
    h = jnp.dot(x_ref[...], wr_ref[...], preferred_element_type=f32)
    h = h + jnp.dot(na, wn_ref[:128], preferred_element_type=f32)
    h = h + jnp.dot(nb, wn_ref[128:], preferred_element_type=f32)
    h = jnp.maximum(h + bc_ref[...], 0.0)
    z = jnp.maximum(jnp.dot(h, w1_ref[...], preferred_element_type=f32)
                    + b1_ref[...], 0.0)
    z = jnp.maximum(jnp.dot(z, w2_ref[...], preferred_element_type=f32)
                    + b2_ref[...], 0.0)
    l = jnp.dot(z, w3_ref[...], preferred_element_type=f32) + b3_ref[...]
    m = jnp.max(l, axis=-1, keepdims=True)
    e = jnp.exp(l - m)
    o_ref[...] = e / jnp.sum(e, axis=-1, keepdims=True)


def _mlp(x, agg, W_root, W_neigh, b_conv, W1, b1, W2, b2, W3, b3):
    B = 2000
    grid = (_N // B,)
    full = lambda shape: pl.BlockSpec(shape, lambda i: tuple(0 for _ in shape))
    return pl.pallas_call(
        _mlp_body,
        grid=grid,
        in_specs=[
            pl.BlockSpec((B, _D), lambda i: (i, 0)),
            pl.BlockSpec((2, B, _TW), lambda i: (0, i, 0)),
            full((_D, _H)),
            full((_D, _H)),
            full((1, _H)),
            full((_H, 400)),
            full((1, 400)),
            full((400, 400)),
            full((1, 400)),
            full((400, 2)),
            full((1, 2)),
        ],
        out_specs=pl.BlockSpec((B, 2), lambda i: (i, 0)),
        out_shape=jax.ShapeDtypeStruct((_N, 2), jnp.float32),
    )(x, agg, W_root, W_neigh, b_conv.reshape(1, _H), W1,
      b1.reshape(1, 400), W2, b2.reshape(1, 400), W3, b3.reshape(1, 2))


@jax.jit
def kernel(x, edge_index, W_root, W_neigh, b_conv, W1, b1, W2, b2, W3, b3):
    ones = jnp.ones((_N, 16), jnp.float32)
    table = jnp.stack([jnp.concatenate([x[:, :128], ones], axis=1),
                       jnp.concatenate([x[:, 128:], ones], axis=1)])
    pad = _EPAD - _E
    srcp = jnp.concatenate(
        [edge_index[0], jnp.zeros((pad,), jnp.int32)]).reshape(
            _NSUB, _NCH, _CHUNK)
    dstp = jnp.concatenate(
        [edge_index[1], jnp.full((pad,), _N, jnp.int32)]).reshape(
            _NSUB, _NCH, _CHUNK)
    agg = _sc_aggregate(table, srcp, dstp)
    return _mlp(x, agg, W_root, W_neigh, b_conv, W1, b1, W2, b2, W3, b3)
